# baseline (speedup 1.0000x reference)
import jax
import jax.numpy as jnp
from jax.experimental import pallas as pl

_H = 1000
_CC = 1000
_LB = 1024


def _body(x_ref, o_ref):
    i = pl.program_id(0)
    idx = x_ref[0, 0, pl.ds(i * _LB, _LB)]
    iota = jax.lax.broadcasted_iota(jnp.int32, (_CC, _LB), 0)
    o_ref[0] = (idx[None, :] == iota).astype(jnp.float32)


def kernel(x):
    b, s = x.shape
    xt = x.T.reshape(s, 1, b).astype(jnp.int32)
    out = pl.pallas_call(
        _body,
        grid=(b // _LB, s),
        in_specs=[pl.BlockSpec((1, 1, b), lambda i, j: (j, 0, 0))],
        out_specs=pl.BlockSpec((1, _CC, _LB), lambda i, j: (j, 0, i)),
        out_shape=jax.ShapeDtypeStruct((s, _H, b), jnp.float32),
    )(xt)
    return jnp.transpose(out, (2, 0, 1))


# speedup vs baseline: 1.0123x; 1.0123x over previous
import jax
import jax.numpy as jnp
from jax.experimental import pallas as pl

_H = 1000
_CC = 1000
_LB = 1024


def _body(x_ref, o_ref):
    i = pl.program_id(1)
    idx = x_ref[0, 0, pl.ds(i * _LB, _LB)]
    iota = jax.lax.broadcasted_iota(jnp.int32, (_CC, _LB), 0)
    o_ref[0] = (idx[None, :] == iota).astype(jnp.float32)


def kernel(x):
    b, s = x.shape
    xt = x.T.reshape(s, 1, b).astype(jnp.int32)
    out = pl.pallas_call(
        _body,
        grid=(s, b // _LB),
        in_specs=[pl.BlockSpec((1, 1, b), lambda j, i: (j, 0, 0))],
        out_specs=pl.BlockSpec((1, _CC, _LB), lambda j, i: (j, 0, i)),
        out_shape=jax.ShapeDtypeStruct((s, _H, b), jnp.float32),
    )(xt)
    return jnp.transpose(out, (2, 0, 1))
